# Initial kernel scaffold; baseline (speedup 1.0000x reference)
import jax
import jax.numpy as jnp
from jax.experimental import pallas as pl


def kernel(features, edge_index, W0, b0, W1, b1, W2, b2):
    raise NotImplementedError("write your pallas kernel here")


# speedup vs baseline: 20.5873x; 20.5873x over previous
import functools

import jax
import jax.numpy as jnp
from jax import lax
from jax.experimental import pallas as pl
from jax.experimental.pallas import tpu as pltpu
from jax.experimental.pallas import tpu_sc as plsc

N = 10000
E = 320000
D = 128

_info = plsc.get_sparse_core_info()
NC = _info.num_cores
NS = _info.num_subcores
L = _info.num_lanes
NW = NC * NS
EPW = E // NW
K = 80
NCHUNK = EPW // K
RPT = N // NS

_mesh = plsc.VectorSubcoreMesh(core_axis_name="c", subcore_axis_name="s")


def _deg_body(src3, dst3, zeros1, dout_hbm, din_hbm,
              sidx, didx, ones_v, dout_sh, din_sh):
    cid = lax.axis_index("c")
    sid = lax.axis_index("s")
    wid = sid * NC + cid

    @pl.when(sid == 0)
    def _():
        pltpu.sync_copy(zeros1, dout_sh)
        pltpu.sync_copy(zeros1, din_sh)

    for j in range(K // L):
        ones_v[pl.ds(j * L, L)] = jnp.full((L,), 1.0, jnp.float32)
    pltpu.sync_copy(src3.at[wid], sidx)
    pltpu.sync_copy(dst3.at[wid], didx)
    plsc.subcore_barrier()

    def chunk(i, carry):
        pltpu.sync_copy(ones_v, dout_sh.at[sidx.at[i]], add=True)
        pltpu.sync_copy(ones_v, din_sh.at[didx.at[i]], add=True)
        return carry

    lax.fori_loop(0, NCHUNK, chunk, 0)
    plsc.subcore_barrier()

    @pl.when(sid == 0)
    def _():
        pltpu.sync_copy(dout_sh, dout_hbm.at[cid])
        pltpu.sync_copy(din_sh, din_hbm.at[cid])


_deg_kernel = pl.kernel(
    _deg_body,
    out_type=(jax.ShapeDtypeStruct((NC, N), jnp.float32),
              jax.ShapeDtypeStruct((NC, N), jnp.float32)),
    mesh=_mesh,
    scratch_types=[
        pltpu.VMEM((NCHUNK, K), jnp.int32),
        pltpu.VMEM((NCHUNK, K), jnp.int32),
        pltpu.VMEM((K,), jnp.float32),
        pltpu.VMEM_SHARED((N,), jnp.float32),
        pltpu.VMEM_SHARED((N,), jnp.float32),
    ],
)


def _agg_body(hs_hbm, srcf, dst3, zeros2, out_hbm,
              sidx, didx, rows0, rows1, accum_sh, sem0, sem1):
    cid = lax.axis_index("c")
    sid = lax.axis_index("s")
    wid = sid * NC + cid
    rbase = sid * RPT

    pltpu.sync_copy(zeros2, accum_sh.at[pl.ds(rbase, RPT)])
    pltpu.sync_copy(srcf.at[pl.ds(wid * EPW, EPW)], sidx)
    pltpu.sync_copy(dst3.at[wid], didx)
    plsc.subcore_barrier()

    def gidx(i):
        return sidx.at[pl.ds(i * K, K)]

    pltpu.async_copy(hs_hbm.at[gidx(0)], rows0, sem0)

    def pair(p, carry):
        a = 2 * p
        b = a + 1
        pltpu.async_copy(hs_hbm.at[gidx(b)], rows1, sem1)
        pltpu.make_async_copy(hs_hbm.at[gidx(a)], rows0, sem0).wait()
        pltpu.sync_copy(rows0, accum_sh.at[didx.at[a]], add=True)
        pltpu.async_copy(hs_hbm.at[gidx(a + 2)], rows0, sem0)
        pltpu.make_async_copy(hs_hbm.at[gidx(b)], rows1, sem1).wait()
        pltpu.sync_copy(rows1, accum_sh.at[didx.at[b]], add=True)
        return carry

    lax.fori_loop(0, (NCHUNK - 1) // 2, pair, 0)
    last = NCHUNK - 1
    pltpu.make_async_copy(hs_hbm.at[gidx(last)], rows0, sem0).wait()
    pltpu.sync_copy(rows0, accum_sh.at[didx.at[last]], add=True)

    plsc.subcore_barrier()
    pltpu.sync_copy(accum_sh.at[pl.ds(rbase, RPT)], out_hbm.at[cid, sid])


_agg_kernel = pl.kernel(
    _agg_body,
    out_type=jax.ShapeDtypeStruct((NC, NS, RPT, D), jnp.float32),
    mesh=_mesh,
    scratch_types=[
        pltpu.VMEM((EPW,), jnp.int32),
        pltpu.VMEM((NCHUNK, K), jnp.int32),
        pltpu.VMEM((K, D), jnp.float32),
        pltpu.VMEM((K, D), jnp.float32),
        pltpu.VMEM_SHARED((N, D), jnp.float32),
        pltpu.SemaphoreType.DMA,
        pltpu.SemaphoreType.DMA,
    ],
)


RB = 2000
GRID = N // RB


def _scale_body(do_ref, di_ref, f_ref, hs_ref, rs_ref, ri_ref):
    do = do_ref[0] + do_ref[1]
    di = di_ref[0] + di_ref[1]
    rs = lax.rsqrt(jnp.maximum(do, 1.0))
    ri = lax.rsqrt(jnp.maximum(di, 1.0))
    rs_ref[...] = rs
    ri_ref[...] = ri
    hs_ref[...] = f_ref[...] * rs


def _scale_call(dout, din, features):
    return pl.pallas_call(
        _scale_body,
        grid=(GRID,),
        in_specs=[
            pl.BlockSpec((NC, RB, 1), lambda i: (0, i, 0)),
            pl.BlockSpec((NC, RB, 1), lambda i: (0, i, 0)),
            pl.BlockSpec((RB, D), lambda i: (i, 0)),
        ],
        out_specs=[
            pl.BlockSpec((RB, D), lambda i: (i, 0)),
            pl.BlockSpec((RB, 1), lambda i: (i, 0)),
            pl.BlockSpec((RB, 1), lambda i: (i, 0)),
        ],
        out_shape=[
            jax.ShapeDtypeStruct((N, D), jnp.float32),
            jax.ShapeDtypeStruct((N, 1), jnp.float32),
            jax.ShapeDtypeStruct((N, 1), jnp.float32),
        ],
    )(dout.reshape(NC, N, 1), din.reshape(NC, N, 1), features)


def _mm_body(apply_rs, p_ref, ri_ref, rs_ref, w_ref, b_ref, o_ref):
    x = (p_ref[0] + p_ref[1]) * ri_ref[...]
    y = jnp.dot(x, w_ref[...], preferred_element_type=jnp.float32)
    y = jnp.maximum(y + b_ref[...], 0.0)
    if apply_rs:
        y = y * rs_ref[...]
    o_ref[...] = y


def _mm_call(p, ri, rs, w, b, apply_rs):
    return pl.pallas_call(
        functools.partial(_mm_body, apply_rs),
        grid=(GRID,),
        in_specs=[
            pl.BlockSpec((NC, RB, D), lambda i: (0, i, 0)),
            pl.BlockSpec((RB, 1), lambda i: (i, 0)),
            pl.BlockSpec((RB, 1), lambda i: (i, 0)),
            pl.BlockSpec((D, D), lambda i: (0, 0)),
            pl.BlockSpec((1, D), lambda i: (0, 0)),
        ],
        out_specs=pl.BlockSpec((RB, D), lambda i: (i, 0)),
        out_shape=jax.ShapeDtypeStruct((N, D), jnp.float32),
    )(p, ri, rs, w, b.reshape(1, D))


def kernel(features, edge_index, W0, b0, W1, b1, W2, b2):
    srcf = edge_index[0]
    src3 = srcf.reshape(NW, NCHUNK, K)
    dst3 = edge_index[1].reshape(NW, NCHUNK, K)
    zeros1 = jnp.zeros((N,), jnp.float32)
    zeros2 = jnp.zeros((RPT, D), jnp.float32)

    dout, din = _deg_kernel(src3, dst3, zeros1)
    hs, rs, ri = _scale_call(dout, din, features)

    for w, b, last in ((W0, b0, False), (W1, b1, False), (W2, b2, True)):
        p = _agg_kernel(hs, srcf, dst3, zeros2).reshape(NC, N, D)
        hs = _mm_call(p, ri, rs, w, b, apply_rs=not last)
    return hs
